# Initial kernel scaffold; baseline (speedup 1.0000x reference)
import jax
import jax.numpy as jnp
from jax.experimental import pallas as pl


def kernel(x, table):
    raise NotImplementedError("write your pallas kernel here")


# speedup vs baseline: 1.8363x; 1.8363x over previous
import functools

import jax
import jax.numpy as jnp
from jax import lax
from jax.experimental import pallas as pl
from jax.experimental.pallas import tpu as pltpu
from jax.experimental.pallas import tpu_sc as plsc

NC = 2
NS = 16
NW = NC * NS

CHUNK = 128


@functools.partial(jax.jit, static_argnames=("V", "D", "B"))
def _gather_rows(idx2d, table, *, V, D, B):
    n_w = B // NW
    n_chunks = n_w // CHUNK
    mesh = plsc.VectorSubcoreMesh(core_axis_name="c", subcore_axis_name="s")

    @functools.partial(
        pl.kernel,
        out_type=jax.ShapeDtypeStruct((B, D), jnp.float32),
        mesh=mesh,
        scratch_types=[
            pltpu.VMEM((n_chunks, CHUNK), jnp.int32),
            pltpu.VMEM((2, CHUNK, D), jnp.float32),
            pltpu.SemaphoreType.DMA((2,)),
        ],
        compiler_params=pltpu.CompilerParams(use_tc_tiling_on_sc=False),
    )
    def k(idx_hbm, table_hbm, out_hbm, idx_v, rows_v, gsem):
        wid = lax.axis_index("s") * NC + lax.axis_index("c")
        base = wid * n_w
        pltpu.sync_copy(idx_hbm.at[pl.ds(wid * n_chunks, n_chunks)], idx_v)

        pltpu.async_copy(table_hbm.at[idx_v.at[0]], rows_v.at[0], gsem.at[0])

        def body(j, _):
            slot = lax.rem(j, 2)
            nslot = lax.rem(j + 1, 2)

            @pl.when(j + 1 < n_chunks)
            def _():
                pltpu.async_copy(
                    table_hbm.at[idx_v.at[j + 1]], rows_v.at[nslot], gsem.at[nslot]
                )

            pltpu.make_async_copy(
                table_hbm.at[idx_v.at[j]], rows_v.at[slot], gsem.at[slot]
            ).wait()
            pltpu.sync_copy(
                rows_v.at[slot], out_hbm.at[pl.ds(base + j * CHUNK, CHUNK)]
            )
            return 0

        lax.fori_loop(0, n_chunks, body, 0)

    return k(idx2d, table)


def kernel(x, table):
    B, T = x.shape
    V, D = table.shape
    n = B * T
    idx2d = x.astype(jnp.int32).reshape(n // CHUNK, CHUNK)
    out = _gather_rows(idx2d, table, V=V, D=D, B=n)
    return out.reshape(B, T, D)


# speedup vs baseline: 1.8740x; 1.0206x over previous
import functools

import jax
import jax.numpy as jnp
from jax import lax
from jax.experimental import pallas as pl
from jax.experimental.pallas import tpu as pltpu
from jax.experimental.pallas import tpu_sc as plsc

NC = 2
NS = 16
NW = NC * NS

CHUNK = 128
NBUF = 8


@functools.partial(jax.jit, static_argnames=("V", "D", "B"))
def _gather_rows(idx2d, table, *, V, D, B):
    n_w = B // NW
    n_chunks = n_w // CHUNK
    mesh = plsc.VectorSubcoreMesh(core_axis_name="c", subcore_axis_name="s")

    n_groups = n_chunks // NBUF

    @functools.partial(
        pl.kernel,
        out_type=jax.ShapeDtypeStruct((B, D), jnp.float32),
        mesh=mesh,
        scratch_types=[
            pltpu.VMEM((n_chunks, CHUNK), jnp.int32),
            pltpu.VMEM((NBUF, CHUNK, D), jnp.float32),
            pltpu.SemaphoreType.DMA((NBUF,)),
            pltpu.SemaphoreType.DMA((NBUF,)),
        ],
        compiler_params=pltpu.CompilerParams(use_tc_tiling_on_sc=False),
    )
    def k(idx_hbm, table_hbm, out_hbm, idx_v, rows_v, gsem, wsem):
        wid = lax.axis_index("s") * NC + lax.axis_index("c")
        base = wid * n_w
        pltpu.sync_copy(idx_hbm.at[pl.ds(wid * n_chunks, n_chunks)], idx_v)

        def gather(j, b):
            pltpu.async_copy(table_hbm.at[idx_v.at[j]], rows_v.at[b], gsem.at[b])

        def wait_gather(j, b):
            pltpu.make_async_copy(
                table_hbm.at[idx_v.at[j]], rows_v.at[b], gsem.at[b]
            ).wait()

        def write(j, b):
            pltpu.async_copy(
                rows_v.at[b], out_hbm.at[pl.ds(base + j * CHUNK, CHUNK)], wsem.at[b]
            )

        def wait_write(j, b):
            pltpu.make_async_copy(
                rows_v.at[b], out_hbm.at[pl.ds(base + j * CHUNK, CHUNK)], wsem.at[b]
            ).wait()

        for b in range(NBUF):
            gather(b, b)

        def body(g, _):
            jg = g * NBUF
            for b in range(NBUF):
                wait_gather(jg + b, b)
                write(jg + b, b)

            @pl.when(g + 1 < n_groups)
            def _():
                for b in range(NBUF):
                    wait_write(jg + b, b)
                    gather(jg + NBUF + b, b)

            return 0

        lax.fori_loop(0, n_groups, body, 0)

        for b in range(NBUF):
            wait_write((n_groups - 1) * NBUF + b, b)

    return k(idx2d, table)


def kernel(x, table):
    B, T = x.shape
    V, D = table.shape
    n = B * T
    idx2d = x.astype(jnp.int32).reshape(n // CHUNK, CHUNK)
    out = _gather_rows(idx2d, table, V=V, D=D, B=n)
    return out.reshape(B, T, D)
